# Initial kernel scaffold; baseline (speedup 1.0000x reference)
import functools

import jax
import jax.numpy as jnp
from jax import lax
from jax.experimental import pallas as pl
from jax.experimental.pallas import tpu as pltpu, tpu_sc as plsc

N = 10000
D = 128
H = 16
C = 40
E = 320000

NC = 2
NS = 16
L = 16
NW = NC * NS

NPAD = 10240
RPT = NPAD // NS
CHUNK = 128
NCHUNK = 80
EPT = NCHUNK * CHUNK
EPAD = EPT * NW
KINF = 8


def _sc_mesh():
    return plsc.VectorSubcoreMesh(core_axis_name="c", subcore_axis_name="s")


def _deg_call(src3, dst3, zrow):
    def body(src_hbm, dst_hbm, z_hbm, dout_hbm, din_hbm,
             sidx, didx, ones_v, acc_out, acc_in):
        cid = lax.axis_index("c")
        sid = lax.axis_index("s")
        w = cid * NS + sid
        sl = pl.ds(sid * RPT, RPT)
        pltpu.sync_copy(z_hbm, acc_out.at[sl])
        pltpu.sync_copy(z_hbm, acc_in.at[sl])
        pltpu.sync_copy(src_hbm.at[w], sidx)
        pltpu.sync_copy(dst_hbm.at[w], didx)
        for i in range(CHUNK // L):
            ones_v[pl.ds(i * L, L)] = jnp.ones((L,), jnp.float32)
        plsc.subcore_barrier()

        def step(c, carry):
            pltpu.sync_copy(ones_v, acc_out.at[sidx.at[c]], add=True)
            pltpu.sync_copy(ones_v, acc_in.at[didx.at[c]], add=True)
            return carry

        lax.fori_loop(0, NCHUNK, step, 0)
        plsc.subcore_barrier()
        pltpu.sync_copy(acc_out.at[sl], dout_hbm.at[cid, sl])
        pltpu.sync_copy(acc_in.at[sl], din_hbm.at[cid, sl])

    f = pl.kernel(
        body,
        out_type=[
            jax.ShapeDtypeStruct((NC, NPAD), jnp.float32),
            jax.ShapeDtypeStruct((NC, NPAD), jnp.float32),
        ],
        mesh=_sc_mesh(),
        scratch_types=[
            pltpu.VMEM((NCHUNK, CHUNK), jnp.int32),
            pltpu.VMEM((NCHUNK, CHUNK), jnp.int32),
            pltpu.VMEM((CHUNK,), jnp.float32),
            pltpu.VMEM_SHARED((NPAD,), jnp.float32),
            pltpu.VMEM_SHARED((NPAD,), jnp.float32),
        ],
    )
    return f(src3, dst3, zrow)


def _edge_call(h, src3, dst3, zrows, F):
    def body(h_hbm, src_hbm, dst_hbm, z_hbm, part_hbm,
             sidx, didx, rows, sem, acc):
        cid = lax.axis_index("c")
        sid = lax.axis_index("s")
        w = cid * NS + sid
        sl = pl.ds(sid * RPT, RPT)
        pltpu.sync_copy(z_hbm, acc.at[sl])
        pltpu.sync_copy(src_hbm.at[w], sidx)
        pltpu.sync_copy(dst_hbm.at[w], didx)
        plsc.subcore_barrier()

        def step(g, carry):
            handles = []
            for b in range(KINF):
                cidx = g * KINF + b
                handles.append(
                    pltpu.async_copy(h_hbm.at[sidx.at[cidx]], rows.at[b], sem))
            for b in range(KINF):
                handles[b].wait()
            for b in range(KINF):
                cidx = g * KINF + b
                pltpu.sync_copy(rows.at[b], acc.at[didx.at[cidx]], add=True)
            return carry

        lax.fori_loop(0, NCHUNK // KINF, step, 0)
        plsc.subcore_barrier()
        pltpu.sync_copy(acc.at[sl], part_hbm.at[cid, sl])

    f = pl.kernel(
        body,
        out_type=jax.ShapeDtypeStruct((NC, NPAD, F), jnp.float32),
        mesh=_sc_mesh(),
        scratch_types=[
            pltpu.VMEM((NCHUNK, CHUNK), jnp.int32),
            pltpu.VMEM((NCHUNK, CHUNK), jnp.int32),
            pltpu.VMEM((KINF, CHUNK, F), jnp.float32),
            pltpu.SemaphoreType.DMA,
            pltpu.VMEM_SHARED((NPAD, F), jnp.float32),
        ],
    )
    return f(h, src3, dst3, zrows)


_BR = 256


def _tc1_call(features, W1, doutp, dinp):
    def body(f_ref, w_ref, do_ref, di_ref, h1_ref, ns_ref, nd_ref):
        dsum_o = do_ref[0] + do_ref[1]
        dsum_i = di_ref[0] + di_ref[1]
        ns = jnp.where(dsum_o > 0, lax.rsqrt(jnp.maximum(dsum_o, 1.0)), 0.0)
        nd = jnp.where(dsum_i > 0, lax.rsqrt(jnp.maximum(dsum_i, 1.0)), 0.0)
        m = jnp.dot(f_ref[...], w_ref[...], preferred_element_type=jnp.float32)
        h1_ref[...] = m * ns
        ns_ref[...] = ns
        nd_ref[...] = nd

    return pl.pallas_call(
        body,
        grid=(NPAD // _BR,),
        in_specs=[
            pl.BlockSpec((_BR, D), lambda i: (i, 0)),
            pl.BlockSpec((D, H), lambda i: (0, 0)),
            pl.BlockSpec((NC, _BR, 1), lambda i: (0, i, 0)),
            pl.BlockSpec((NC, _BR, 1), lambda i: (0, i, 0)),
        ],
        out_specs=[
            pl.BlockSpec((_BR, H), lambda i: (i, 0)),
            pl.BlockSpec((_BR, 1), lambda i: (i, 0)),
            pl.BlockSpec((_BR, 1), lambda i: (i, 0)),
        ],
        out_shape=[
            jax.ShapeDtypeStruct((NPAD, H), jnp.float32),
            jax.ShapeDtypeStruct((NPAD, 1), jnp.float32),
            jax.ShapeDtypeStruct((NPAD, 1), jnp.float32),
        ],
    )(features, W1, doutp, dinp)


def _tc2_call(p, nd, ns, b1r, W2):
    def body(p_ref, nd_ref, ns_ref, b1_ref, w2_ref, h2_ref):
        agg = p_ref[0] + p_ref[1]
        x = jnp.maximum(agg * nd_ref[...] + b1_ref[...], 0.0)
        h2 = jnp.dot(x, w2_ref[...], preferred_element_type=jnp.float32)
        h2_ref[...] = h2 * ns_ref[...]

    return pl.pallas_call(
        body,
        grid=(NPAD // _BR,),
        in_specs=[
            pl.BlockSpec((NC, _BR, H), lambda i: (0, i, 0)),
            pl.BlockSpec((_BR, 1), lambda i: (i, 0)),
            pl.BlockSpec((_BR, 1), lambda i: (i, 0)),
            pl.BlockSpec((1, H), lambda i: (0, 0)),
            pl.BlockSpec((H, C), lambda i: (0, 0)),
        ],
        out_specs=pl.BlockSpec((_BR, C), lambda i: (i, 0)),
        out_shape=jax.ShapeDtypeStruct((NPAD, C), jnp.float32),
    )(p, nd, ns, b1r, W2)


def _tc3_call(q, nd, b2r):
    def body(q_ref, nd_ref, b2_ref, o_ref):
        o_ref[...] = (q_ref[0] + q_ref[1]) * nd_ref[...] + b2_ref[...]

    return pl.pallas_call(
        body,
        grid=(NPAD // _BR,),
        in_specs=[
            pl.BlockSpec((NC, _BR, C), lambda i: (0, i, 0)),
            pl.BlockSpec((_BR, 1), lambda i: (i, 0)),
            pl.BlockSpec((1, C), lambda i: (0, 0)),
        ],
        out_specs=pl.BlockSpec((_BR, C), lambda i: (i, 0)),
        out_shape=jax.ShapeDtypeStruct((NPAD, C), jnp.float32),
    )(q, nd, b2r)


def kernel(features, edge_index, W1, b1, W2, b2):
    src = edge_index[0]
    dst = edge_index[1]
    pad = N + (jnp.arange(EPAD - E, dtype=jnp.int32) % (NPAD - N))
    src3 = jnp.concatenate([src, pad]).reshape(NW, NCHUNK, CHUNK)
    dst3 = jnp.concatenate([dst, pad]).reshape(NW, NCHUNK, CHUNK)

    dout, din = _deg_call(src3, dst3, jnp.zeros((RPT,), jnp.float32))
    h1, ns, nd = _tc1_call(features, W1,
                           dout.reshape(NC, NPAD, 1), din.reshape(NC, NPAD, 1))
    p = _edge_call(h1, src3, dst3, jnp.zeros((RPT, H), jnp.float32), H)
    h2 = _tc2_call(p, nd, ns, b1.reshape(1, H), W2)
    q = _edge_call(h2, src3, dst3, jnp.zeros((RPT, C), jnp.float32), C)
    out = _tc3_call(q, nd, b2.reshape(1, C))
    return out[:N]


# speedup vs baseline: 11.6214x; 11.6214x over previous
import functools

import jax
import jax.numpy as jnp
from jax import lax
from jax.experimental import pallas as pl
from jax.experimental.pallas import tpu as pltpu, tpu_sc as plsc

N = 10000
D = 128
H = 16
C = 40
E = 320000

NC = 2
NS = 16
L = 16
NW = NC * NS

NPAD = 10240
RPT = NPAD // NS
CHUNK = 128
NCHUNK = 80
EPT = NCHUNK * CHUNK
EPAD = EPT * NW
KINF = 8


def _sc_mesh():
    return plsc.VectorSubcoreMesh(core_axis_name="c", subcore_axis_name="s")


def _deg_call(src3, dst3, zrow):
    def body(src_hbm, dst_hbm, z_hbm, dout_hbm, din_hbm,
             sidx, didx, ones_v, acc_out, acc_in):
        cid = lax.axis_index("c")
        sid = lax.axis_index("s")
        w = cid * NS + sid
        sl = pl.ds(sid * RPT, RPT)
        pltpu.sync_copy(z_hbm, acc_out.at[sl])
        pltpu.sync_copy(z_hbm, acc_in.at[sl])
        pltpu.sync_copy(src_hbm.at[w], sidx)
        pltpu.sync_copy(dst_hbm.at[w], didx)
        for i in range(CHUNK // L):
            ones_v[pl.ds(i * L, L)] = jnp.ones((L,), jnp.float32)
        plsc.subcore_barrier()

        def step(c, carry):
            pltpu.sync_copy(ones_v, acc_out.at[sidx.at[c]], add=True)
            pltpu.sync_copy(ones_v, acc_in.at[didx.at[c]], add=True)
            return carry

        lax.fori_loop(0, NCHUNK, step, 0)
        plsc.subcore_barrier()
        pltpu.sync_copy(acc_out.at[sl], dout_hbm.at[cid, sl])
        pltpu.sync_copy(acc_in.at[sl], din_hbm.at[cid, sl])

    f = pl.kernel(
        body,
        out_type=[
            jax.ShapeDtypeStruct((NC, NPAD), jnp.float32),
            jax.ShapeDtypeStruct((NC, NPAD), jnp.float32),
        ],
        mesh=_sc_mesh(),
        scratch_types=[
            pltpu.VMEM((NCHUNK, CHUNK), jnp.int32),
            pltpu.VMEM((NCHUNK, CHUNK), jnp.int32),
            pltpu.VMEM((CHUNK,), jnp.float32),
            pltpu.VMEM_SHARED((NPAD,), jnp.float32),
            pltpu.VMEM_SHARED((NPAD,), jnp.float32),
        ],
    )
    return f(src3, dst3, zrow)


def _edge_call(h, src3, dst3, zrows, F):
    def body(h_hbm, src_hbm, dst_hbm, z_hbm, part_hbm,
             sidx, didx, rows, sem, acc):
        cid = lax.axis_index("c")
        sid = lax.axis_index("s")
        w = cid * NS + sid
        sl = pl.ds(sid * RPT, RPT)
        pltpu.sync_copy(z_hbm, acc.at[sl])
        pltpu.sync_copy(src_hbm.at[w], sidx)
        pltpu.sync_copy(dst_hbm.at[w], didx)
        plsc.subcore_barrier()

        def step(g, carry):
            handles = []
            for b in range(KINF):
                cidx = g * KINF + b
                handles.append(
                    pltpu.async_copy(h_hbm.at[sidx.at[cidx]], rows.at[b], sem))
            for b in range(KINF):
                handles[b].wait()
            for b in range(KINF):
                cidx = g * KINF + b
                pltpu.sync_copy(rows.at[b], acc.at[didx.at[cidx]], add=True)
            return carry

        lax.fori_loop(0, NCHUNK // KINF, step, 0)
        plsc.subcore_barrier()
        pltpu.sync_copy(acc.at[sl], part_hbm.at[cid, sl])

    f = pl.kernel(
        body,
        out_type=jax.ShapeDtypeStruct((NC, NPAD, F), jnp.float32),
        mesh=_sc_mesh(),
        compiler_params=pltpu.CompilerParams(use_tc_tiling_on_sc=False),
        scratch_types=[
            pltpu.VMEM((NCHUNK, CHUNK), jnp.int32),
            pltpu.VMEM((NCHUNK, CHUNK), jnp.int32),
            pltpu.VMEM((KINF, CHUNK, F), jnp.float32),
            pltpu.SemaphoreType.DMA,
            pltpu.VMEM_SHARED((NPAD, F), jnp.float32),
        ],
    )
    return f(h, src3, dst3, zrows)


_BR = 256


def _tc1_call(features, W1, doutp, dinp):
    def body(f_ref, w_ref, do_ref, di_ref, h1_ref, ns_ref, nd_ref):
        dsum_o = do_ref[0] + do_ref[1]
        dsum_i = di_ref[0] + di_ref[1]
        ns = jnp.where(dsum_o > 0, lax.rsqrt(jnp.maximum(dsum_o, 1.0)), 0.0)
        nd = jnp.where(dsum_i > 0, lax.rsqrt(jnp.maximum(dsum_i, 1.0)), 0.0)
        m = jnp.dot(f_ref[...], w_ref[...], preferred_element_type=jnp.float32)
        h1_ref[...] = m * ns
        ns_ref[...] = ns
        nd_ref[...] = nd

    return pl.pallas_call(
        body,
        grid=(NPAD // _BR,),
        in_specs=[
            pl.BlockSpec((_BR, D), lambda i: (i, 0)),
            pl.BlockSpec((D, H), lambda i: (0, 0)),
            pl.BlockSpec((NC, _BR, 1), lambda i: (0, i, 0)),
            pl.BlockSpec((NC, _BR, 1), lambda i: (0, i, 0)),
        ],
        out_specs=[
            pl.BlockSpec((_BR, H), lambda i: (i, 0)),
            pl.BlockSpec((_BR, 1), lambda i: (i, 0)),
            pl.BlockSpec((_BR, 1), lambda i: (i, 0)),
        ],
        out_shape=[
            jax.ShapeDtypeStruct((NPAD, H), jnp.float32),
            jax.ShapeDtypeStruct((NPAD, 1), jnp.float32),
            jax.ShapeDtypeStruct((NPAD, 1), jnp.float32),
        ],
    )(features, W1, doutp, dinp)


def _tc2_call(p, nd, ns, b1r, W2):
    def body(p_ref, nd_ref, ns_ref, b1_ref, w2_ref, h2_ref):
        agg = p_ref[0] + p_ref[1]
        x = jnp.maximum(agg * nd_ref[...] + b1_ref[...], 0.0)
        h2 = jnp.dot(x, w2_ref[...], preferred_element_type=jnp.float32)
        h2_ref[...] = h2 * ns_ref[...]

    return pl.pallas_call(
        body,
        grid=(NPAD // _BR,),
        in_specs=[
            pl.BlockSpec((NC, _BR, H), lambda i: (0, i, 0)),
            pl.BlockSpec((_BR, 1), lambda i: (i, 0)),
            pl.BlockSpec((_BR, 1), lambda i: (i, 0)),
            pl.BlockSpec((1, H), lambda i: (0, 0)),
            pl.BlockSpec((H, C), lambda i: (0, 0)),
        ],
        out_specs=pl.BlockSpec((_BR, C), lambda i: (i, 0)),
        out_shape=jax.ShapeDtypeStruct((NPAD, C), jnp.float32),
    )(p, nd, ns, b1r, W2)


def _tc3_call(q, nd, b2r):
    def body(q_ref, nd_ref, b2_ref, o_ref):
        o_ref[...] = (q_ref[0] + q_ref[1]) * nd_ref[...] + b2_ref[...]

    return pl.pallas_call(
        body,
        grid=(NPAD // _BR,),
        in_specs=[
            pl.BlockSpec((NC, _BR, C), lambda i: (0, i, 0)),
            pl.BlockSpec((_BR, 1), lambda i: (i, 0)),
            pl.BlockSpec((1, C), lambda i: (0, 0)),
        ],
        out_specs=pl.BlockSpec((_BR, C), lambda i: (i, 0)),
        out_shape=jax.ShapeDtypeStruct((NPAD, C), jnp.float32),
    )(q, nd, b2r)


def kernel(features, edge_index, W1, b1, W2, b2):
    src = edge_index[0]
    dst = edge_index[1]
    pad = N + (jnp.arange(EPAD - E, dtype=jnp.int32) % (NPAD - N))
    src3 = jnp.concatenate([src, pad]).reshape(NW, NCHUNK, CHUNK)
    dst3 = jnp.concatenate([dst, pad]).reshape(NW, NCHUNK, CHUNK)

    dout, din = _deg_call(src3, dst3, jnp.zeros((RPT,), jnp.float32))
    h1, ns, nd = _tc1_call(features, W1,
                           dout.reshape(NC, NPAD, 1), din.reshape(NC, NPAD, 1))
    p = _edge_call(h1, src3, dst3, jnp.zeros((RPT, H), jnp.float32), H)
    h2 = _tc2_call(p, nd, ns, b1.reshape(1, H), W2)
    q = _edge_call(h2, src3, dst3, jnp.zeros((RPT, C), jnp.float32), C)
    out = _tc3_call(q, nd, b2.reshape(1, C))
    return out[:N]


# speedup vs baseline: 14.0309x; 1.2073x over previous
import jax
import jax.numpy as jnp
from jax import lax
from jax.experimental import pallas as pl
from jax.experimental.pallas import tpu as pltpu, tpu_sc as plsc

N = 10000
D = 128
H = 16
C = 40
E = 320000

NC = 2
NS = 16
L = 16
NW = NC * NS

NPAD = 10240
RPT = NPAD // NS
CHUNK = 125
NCHUNK = 80
KINF = 4
NG = NCHUNK // KINF

BR = 1024
NBLK = NPAD // BR


def _sc_mesh():
    return plsc.VectorSubcoreMesh(core_axis_name="c", subcore_axis_name="s")


def _deg_call(edges, zrow):
    def body(e_hbm, z_hbm, dout_hbm, din_hbm,
             sidx, didx, ones_v, sem, acc_out, acc_in):
        cid = lax.axis_index("c")
        sid = lax.axis_index("s")
        w = cid * NS + sid
        sl = pl.ds(sid * RPT, RPT)
        pltpu.sync_copy(z_hbm, acc_out.at[sl])
        pltpu.sync_copy(z_hbm, acc_in.at[sl])
        pltpu.sync_copy(e_hbm.at[0, w], sidx)
        pltpu.sync_copy(e_hbm.at[1, w], didx)
        for i in range(CHUNK // L + 1):
            o = min(i * L, CHUNK - L)
            ones_v[pl.ds(o, L)] = jnp.ones((L,), jnp.float32)
        plsc.subcore_barrier()

        def step(c, carry):
            ha = pltpu.async_copy(ones_v, acc_out.at[sidx.at[c]], sem,
                                  add=True)
            hb = pltpu.async_copy(ones_v, acc_in.at[didx.at[c]], sem,
                                  add=True)
            ha.wait()
            hb.wait()
            return carry

        lax.fori_loop(0, NCHUNK, step, 0)
        plsc.subcore_barrier()
        pltpu.sync_copy(acc_out.at[sl], dout_hbm.at[cid, sl])
        pltpu.sync_copy(acc_in.at[sl], din_hbm.at[cid, sl])

    f = pl.kernel(
        body,
        out_type=[
            jax.ShapeDtypeStruct((NC, NPAD), jnp.float32),
            jax.ShapeDtypeStruct((NC, NPAD), jnp.float32),
        ],
        mesh=_sc_mesh(),
        compiler_params=pltpu.CompilerParams(use_tc_tiling_on_sc=False),
        scratch_types=[
            pltpu.VMEM((NCHUNK, CHUNK), jnp.int32),
            pltpu.VMEM((NCHUNK, CHUNK), jnp.int32),
            pltpu.VMEM((CHUNK,), jnp.float32),
            pltpu.SemaphoreType.DMA,
            pltpu.VMEM_SHARED((NPAD,), jnp.float32),
            pltpu.VMEM_SHARED((NPAD,), jnp.float32),
        ],
    )
    return f(edges, zrow)


def _edge_call(h, edges, zrows, F):
    def body(h_hbm, e_hbm, z_hbm, part_hbm,
             sidx, didx, rows, semg, sems, acc):
        cid = lax.axis_index("c")
        sid = lax.axis_index("s")
        w = cid * NS + sid
        sl = pl.ds(sid * RPT, RPT)
        pltpu.sync_copy(z_hbm, acc.at[sl])
        pltpu.sync_copy(e_hbm.at[0, w], sidx)
        pltpu.sync_copy(e_hbm.at[1, w], didx)
        plsc.subcore_barrier()

        def fire_gathers(g, pg):
            for b in range(KINF):
                pltpu.async_copy(
                    h_hbm.at[sidx.at[g * KINF + b]], rows.at[pg, b], semg)

        def drain_gathers(g, pg):
            for b in range(KINF):
                pltpu.make_async_copy(
                    h_hbm.at[sidx.at[g * KINF + b]], rows.at[pg, b],
                    semg).wait()

        def fire_scatters(g, pg):
            for b in range(KINF):
                pltpu.async_copy(
                    rows.at[pg, b], acc.at[didx.at[g * KINF + b]], sems,
                    add=True)

        def drain_scatters(g, pg):
            for b in range(KINF):
                pltpu.make_async_copy(
                    rows.at[pg, b], acc.at[didx.at[g * KINF + b]],
                    sems).wait()

        fire_gathers(0, 0)
        drain_gathers(0, 0)

        def step(g, carry):
            pg = lax.rem(g, 2)

            @pl.when(g > 0)
            def _():
                drain_scatters(g, 1 - pg)

            fire_scatters(g, pg)

            @pl.when(g < NG - 1)
            def _():
                fire_gathers(g + 1, 1 - pg)
                drain_gathers(g + 1, 1 - pg)

            return carry

        lax.fori_loop(0, NG, step, 0)
        drain_scatters(NG - 1, lax.rem(NG - 1, 2))
        plsc.subcore_barrier()
        pltpu.sync_copy(acc.at[sl], part_hbm.at[cid, sl])

    f = pl.kernel(
        body,
        out_type=jax.ShapeDtypeStruct((NC, NPAD, F), jnp.float32),
        mesh=_sc_mesh(),
        compiler_params=pltpu.CompilerParams(use_tc_tiling_on_sc=False),
        scratch_types=[
            pltpu.VMEM((NCHUNK, CHUNK), jnp.int32),
            pltpu.VMEM((NCHUNK, CHUNK), jnp.int32),
            pltpu.VMEM((2, KINF, CHUNK, F), jnp.float32),
            pltpu.SemaphoreType.DMA,
            pltpu.SemaphoreType.DMA,
            pltpu.VMEM_SHARED((NPAD, F), jnp.float32),
        ],
    )
    return f(h, edges, zrows)


def _norms_call(do0, do1, di0, di1):
    def body(do0_ref, do1_ref, di0_ref, di1_ref, ns_ref, nd_ref):
        dsum_o = do0_ref[...] + do1_ref[...]
        dsum_i = di0_ref[...] + di1_ref[...]
        ns = jnp.where(dsum_o > 0, lax.rsqrt(jnp.maximum(dsum_o, 1.0)), 0.0)
        nd = jnp.where(dsum_i > 0, lax.rsqrt(jnp.maximum(dsum_i, 1.0)), 0.0)
        ns_ref[...] = ns.reshape(BR)
        nd_ref[...] = nd.reshape(BR)

    nrow = BR // 128
    return pl.pallas_call(
        body,
        grid=(NBLK,),
        in_specs=[
            pl.BlockSpec((nrow, 128), lambda i: (i, 0)),
            pl.BlockSpec((nrow, 128), lambda i: (i, 0)),
            pl.BlockSpec((nrow, 128), lambda i: (i, 0)),
            pl.BlockSpec((nrow, 128), lambda i: (i, 0)),
        ],
        out_specs=[
            pl.BlockSpec((BR,), lambda i: (i,)),
            pl.BlockSpec((BR,), lambda i: (i,)),
        ],
        out_shape=[
            jax.ShapeDtypeStruct((NPAD,), jnp.float32),
            jax.ShapeDtypeStruct((NPAD,), jnp.float32),
        ],
    )(do0, do1, di0, di1)


def _tc1_call(features, W1, ns):
    def body(f_ref, w_ref, ns_ref, h1_ref):
        nsv = ns_ref[...].reshape(BR, 1)
        m = jnp.dot(f_ref[...], w_ref[...], preferred_element_type=jnp.float32)
        h1_ref[...] = m * nsv

    return pl.pallas_call(
        body,
        grid=(NBLK,),
        in_specs=[
            pl.BlockSpec((BR, D), lambda i: (i, 0)),
            pl.BlockSpec((D, H), lambda i: (0, 0)),
            pl.BlockSpec((BR,), lambda i: (i,)),
        ],
        out_specs=pl.BlockSpec((BR, H), lambda i: (i, 0)),
        out_shape=jax.ShapeDtypeStruct((NPAD, H), jnp.float32),
    )(features, W1, ns)


def _tc2_call(p0, p1, nd, ns, b1r, W2):
    def body(p0_ref, p1_ref, nd_ref, ns_ref, b1_ref, w2_ref, h2_ref):
        pf = p0_ref[...] + p1_ref[...]
        parts = [pf[:, a * H:(a + 1) * H] for a in range(128 // H)]
        agg = jnp.stack(parts, axis=1).reshape(BR, H)
        ndv = nd_ref[...].reshape(BR, 1)
        nsv = ns_ref[...].reshape(BR, 1)
        x = jnp.maximum(agg * ndv + b1_ref[...], 0.0)
        h2 = jnp.dot(x, w2_ref[...], preferred_element_type=jnp.float32)
        h2_ref[...] = h2 * nsv

    nrow = BR // 128
    return pl.pallas_call(
        body,
        grid=(NBLK,),
        in_specs=[
            pl.BlockSpec((BR * H // 128, 128), lambda i: (i, 0)),
            pl.BlockSpec((BR * H // 128, 128), lambda i: (i, 0)),
            pl.BlockSpec((BR,), lambda i: (i,)),
            pl.BlockSpec((BR,), lambda i: (i,)),
            pl.BlockSpec((1, H), lambda i: (0, 0)),
            pl.BlockSpec((H, C), lambda i: (0, 0)),
        ],
        out_specs=pl.BlockSpec((BR, C), lambda i: (i, 0)),
        out_shape=jax.ShapeDtypeStruct((NPAD, C), jnp.float32),
    )(p0, p1, nd, ns, b1r, W2)


def _tc3_call(q0, q1, ndrep, b2p):
    def body(q0_ref, q1_ref, ndr_ref, b2_ref, o_ref):
        o_ref[...] = (q0_ref[...] + q1_ref[...]) * ndr_ref[...] + b2_ref[...]

    rows = BR * C // 128
    return pl.pallas_call(
        body,
        grid=(NBLK,),
        in_specs=[
            pl.BlockSpec((rows, 128), lambda i: (i, 0)),
            pl.BlockSpec((rows, 128), lambda i: (i, 0)),
            pl.BlockSpec((rows, 128), lambda i: (i, 0)),
            pl.BlockSpec((rows, 128), lambda i: (0, 0)),
        ],
        out_specs=pl.BlockSpec((rows, 128), lambda i: (i, 0)),
        out_shape=jax.ShapeDtypeStruct((NPAD * C // 128, 128), jnp.float32),
    )(q0, q1, ndrep, b2p)


def kernel(features, edge_index, W1, b1, W2, b2):
    edges = edge_index.reshape(2, NW, NCHUNK, CHUNK)

    dout, din = _deg_call(edges, jnp.zeros((RPT,), jnp.float32))
    do0 = dout[0].reshape(NPAD // 128, 128)
    do1 = dout[1].reshape(NPAD // 128, 128)
    di0 = din[0].reshape(NPAD // 128, 128)
    di1 = din[1].reshape(NPAD // 128, 128)

    ns, nd = _norms_call(do0, do1, di0, di1)
    h1 = _tc1_call(features, W1, ns)
    p = _edge_call(h1, edges, jnp.zeros((RPT, H), jnp.float32), H)
    h2c = _tc2_call(p[0].reshape(NPAD * H // 128, 128),
                    p[1].reshape(NPAD * H // 128, 128),
                    nd, ns, b1.reshape(1, H), W2)
    q = _edge_call(h2c, edges,
                   jnp.zeros((RPT, C), jnp.float32), C)
    ndrep = jnp.repeat(nd, C).reshape(NPAD * C // 128, 128)
    b2p = jnp.tile(b2, BR).reshape(BR * C // 128, 128)
    outp = _tc3_call(q[0].reshape(NPAD * C // 128, 128),
                     q[1].reshape(NPAD * C // 128, 128),
                     ndrep, b2p)
    return outp.reshape(NPAD, C)[:N]


# speedup vs baseline: 18.4500x; 1.3150x over previous
import jax
import jax.numpy as jnp
from jax import lax
from jax.experimental import pallas as pl
from jax.experimental.pallas import tpu as pltpu, tpu_sc as plsc

N = 10000
D = 128
H = 16
C = 40
E = 320000

NC = 2
NS = 16
L = 16
NW = NC * NS

NPAD = 10240
RPT = NPAD // NS
CHUNK = 125
NCHUNK = 80
KINF = 4
NG = NCHUNK // KINF

BR = 1024
NBLK = NPAD // BR


def _sc_mesh():
    return plsc.VectorSubcoreMesh(core_axis_name="c", subcore_axis_name="s")


def _deg_call(edges, zrow):
    def body(e_hbm, z_hbm, dout0_hbm, dout1_hbm, din0_hbm, din1_hbm,
             sidx, didx, ones_v, sem, acc_out, acc_in):
        cid = lax.axis_index("c")
        sid = lax.axis_index("s")
        w = cid * NS + sid
        sl = pl.ds(sid * RPT, RPT)
        pltpu.sync_copy(z_hbm, acc_out.at[sl])
        pltpu.sync_copy(z_hbm, acc_in.at[sl])
        pltpu.sync_copy(e_hbm.at[0, w], sidx)
        pltpu.sync_copy(e_hbm.at[1, w], didx)
        for i in range(CHUNK // L + 1):
            o = min(i * L, CHUNK - L)
            ones_v[pl.ds(o, L)] = jnp.ones((L,), jnp.float32)
        plsc.subcore_barrier()

        def step(c, carry):
            ha = pltpu.async_copy(ones_v, acc_out.at[sidx.at[c]], sem,
                                  add=True)
            hb = pltpu.async_copy(ones_v, acc_in.at[didx.at[c]], sem,
                                  add=True)
            ha.wait()
            hb.wait()
            return carry

        lax.fori_loop(0, NCHUNK, step, 0)
        plsc.subcore_barrier()

        @pl.when(cid == 0)
        def _():
            pltpu.sync_copy(acc_out.at[sl], dout0_hbm.at[sl])
            pltpu.sync_copy(acc_in.at[sl], din0_hbm.at[sl])

        @pl.when(cid == 1)
        def _():
            pltpu.sync_copy(acc_out.at[sl], dout1_hbm.at[sl])
            pltpu.sync_copy(acc_in.at[sl], din1_hbm.at[sl])

    f = pl.kernel(
        body,
        out_type=[jax.ShapeDtypeStruct((NPAD,), jnp.float32)] * 4,
        mesh=_sc_mesh(),
        compiler_params=pltpu.CompilerParams(use_tc_tiling_on_sc=False),
        scratch_types=[
            pltpu.VMEM((NCHUNK, CHUNK), jnp.int32),
            pltpu.VMEM((NCHUNK, CHUNK), jnp.int32),
            pltpu.VMEM((CHUNK,), jnp.float32),
            pltpu.SemaphoreType.DMA,
            pltpu.VMEM_SHARED((NPAD,), jnp.float32),
            pltpu.VMEM_SHARED((NPAD,), jnp.float32),
        ],
    )
    return f(edges, zrow)


def _edge_call(h, edges, zrows, F):
    def body(h_hbm, e_hbm, z_hbm, part0_hbm, part1_hbm,
             sidx, didx, rows, semg, sems, acc):
        cid = lax.axis_index("c")
        sid = lax.axis_index("s")
        w = cid * NS + sid
        sl = pl.ds(sid * RPT, RPT)
        pltpu.sync_copy(z_hbm, acc.at[sl])
        pltpu.sync_copy(e_hbm.at[0, w], sidx)
        pltpu.sync_copy(e_hbm.at[1, w], didx)
        plsc.subcore_barrier()

        def fire_gathers(g, pg):
            for b in range(KINF):
                pltpu.async_copy(
                    h_hbm.at[sidx.at[g * KINF + b]], rows.at[pg, b], semg)

        def drain_gathers(g, pg):
            for b in range(KINF):
                pltpu.make_async_copy(
                    h_hbm.at[sidx.at[g * KINF + b]], rows.at[pg, b],
                    semg).wait()

        def fire_scatters(g, pg):
            for b in range(KINF):
                pltpu.async_copy(
                    rows.at[pg, b], acc.at[didx.at[g * KINF + b]], sems,
                    add=True)

        def drain_scatters(g, pg):
            for b in range(KINF):
                pltpu.make_async_copy(
                    rows.at[pg, b], acc.at[didx.at[g * KINF + b]],
                    sems).wait()

        fire_gathers(0, 0)
        drain_gathers(0, 0)

        def step(g, carry):
            pg = lax.rem(g, 2)

            @pl.when(g > 0)
            def _():
                drain_scatters(g, 1 - pg)

            fire_scatters(g, pg)

            @pl.when(g < NG - 1)
            def _():
                fire_gathers(g + 1, 1 - pg)
                drain_gathers(g + 1, 1 - pg)

            return carry

        lax.fori_loop(0, NG, step, 0)
        drain_scatters(NG - 1, lax.rem(NG - 1, 2))
        plsc.subcore_barrier()

        @pl.when(cid == 0)
        def _():
            pltpu.sync_copy(acc.at[sl], part0_hbm.at[sl])

        @pl.when(cid == 1)
        def _():
            pltpu.sync_copy(acc.at[sl], part1_hbm.at[sl])

    f = pl.kernel(
        body,
        out_type=[jax.ShapeDtypeStruct((NPAD, F), jnp.float32)] * 2,
        mesh=_sc_mesh(),
        compiler_params=pltpu.CompilerParams(use_tc_tiling_on_sc=False),
        scratch_types=[
            pltpu.VMEM((NCHUNK, CHUNK), jnp.int32),
            pltpu.VMEM((NCHUNK, CHUNK), jnp.int32),
            pltpu.VMEM((2, KINF, CHUNK, F), jnp.float32),
            pltpu.SemaphoreType.DMA,
            pltpu.SemaphoreType.DMA,
            pltpu.VMEM_SHARED((NPAD, F), jnp.float32),
        ],
    )
    return f(h, edges, zrows)


def _norms_call(do0, do1, di0, di1):
    def body(do0_ref, do1_ref, di0_ref, di1_ref, ns_ref, nd_ref):
        dsum_o = do0_ref[...] + do1_ref[...]
        dsum_i = di0_ref[...] + di1_ref[...]
        ns_ref[...] = jnp.where(
            dsum_o > 0, lax.rsqrt(jnp.maximum(dsum_o, 1.0)), 0.0)
        nd_ref[...] = jnp.where(
            dsum_i > 0, lax.rsqrt(jnp.maximum(dsum_i, 1.0)), 0.0)

    return pl.pallas_call(
        body,
        out_shape=[
            jax.ShapeDtypeStruct((NPAD,), jnp.float32),
            jax.ShapeDtypeStruct((NPAD,), jnp.float32),
        ],
    )(do0, do1, di0, di1)


def _tc1_call(features, W1, ns):
    def body(f_ref, w_ref, ns_ref, h1_ref):
        nsv = ns_ref[...].reshape(BR, 1)
        m = jnp.dot(f_ref[...], w_ref[...], preferred_element_type=jnp.float32)
        h1_ref[...] = m * nsv

    return pl.pallas_call(
        body,
        grid=(NBLK,),
        in_specs=[
            pl.BlockSpec((BR, D), lambda i: (i, 0)),
            pl.BlockSpec((D, H), lambda i: (0, 0)),
            pl.BlockSpec((BR,), lambda i: (i,)),
        ],
        out_specs=pl.BlockSpec((BR, H), lambda i: (i, 0)),
        out_shape=jax.ShapeDtypeStruct((NPAD, H), jnp.float32),
    )(features, W1, ns)


def _tc2_call(p0, p1, nd, ns, b1r, W2):
    def body(p0_ref, p1_ref, nd_ref, ns_ref, b1_ref, w2_ref, h2_ref):
        pf = p0_ref[...] + p1_ref[...]
        parts = [pf[:, a * H:(a + 1) * H] for a in range(128 // H)]
        agg = jnp.stack(parts, axis=1).reshape(BR, H)
        ndv = nd_ref[...].reshape(BR, 1)
        nsv = ns_ref[...].reshape(BR, 1)
        x = jnp.maximum(agg * ndv + b1_ref[...], 0.0)
        h2 = jnp.dot(x, w2_ref[...], preferred_element_type=jnp.float32)
        h2_ref[...] = h2 * nsv

    nrow = BR // 128
    return pl.pallas_call(
        body,
        grid=(NBLK,),
        in_specs=[
            pl.BlockSpec((BR * H // 128, 128), lambda i: (i, 0)),
            pl.BlockSpec((BR * H // 128, 128), lambda i: (i, 0)),
            pl.BlockSpec((BR,), lambda i: (i,)),
            pl.BlockSpec((BR,), lambda i: (i,)),
            pl.BlockSpec((1, H), lambda i: (0, 0)),
            pl.BlockSpec((H, C), lambda i: (0, 0)),
        ],
        out_specs=pl.BlockSpec((BR, C), lambda i: (i, 0)),
        out_shape=jax.ShapeDtypeStruct((NPAD, C), jnp.float32),
    )(p0, p1, nd, ns, b1r, W2)


def _tc3_call(q0, q1, ndrep, b2p):
    def body(q0_ref, q1_ref, ndr_ref, b2_ref, o_ref):
        o_ref[...] = (q0_ref[...] + q1_ref[...]) * ndr_ref[...] + b2_ref[...]

    rows = BR * C // 128
    return pl.pallas_call(
        body,
        grid=(NBLK,),
        in_specs=[
            pl.BlockSpec((rows, 128), lambda i: (i, 0)),
            pl.BlockSpec((rows, 128), lambda i: (i, 0)),
            pl.BlockSpec((rows, 128), lambda i: (i, 0)),
            pl.BlockSpec((rows, 128), lambda i: (0, 0)),
        ],
        out_specs=pl.BlockSpec((rows, 128), lambda i: (i, 0)),
        out_shape=jax.ShapeDtypeStruct((N * C // 128, 128), jnp.float32),
    )(q0, q1, ndrep, b2p)


def kernel(features, edge_index, W1, b1, W2, b2):
    edges = edge_index.reshape(2, NW, NCHUNK, CHUNK)

    do0, do1, di0, di1 = _deg_call(edges, jnp.zeros((RPT,), jnp.float32))
    ns, nd = _norms_call(do0, do1, di0, di1)
    h1 = _tc1_call(features, W1, ns)
    p0, p1 = _edge_call(h1, edges, jnp.zeros((RPT, H), jnp.float32), H)
    h2 = _tc2_call(p0.reshape(NPAD * H // 128, 128),
                   p1.reshape(NPAD * H // 128, 128),
                   nd, ns, b1.reshape(1, H), W2)
    q0, q1 = _edge_call(h2, edges, jnp.zeros((RPT, C), jnp.float32), C)
    ndrep = jnp.repeat(nd, C).reshape(NPAD * C // 128, 128)
    b2p = jnp.tile(b2, BR).reshape(BR * C // 128, 128)
    outp = _tc3_call(q0.reshape(NPAD * C // 128, 128),
                     q1.reshape(NPAD * C // 128, 128),
                     ndrep, b2p)
    return outp.reshape(N, C)


# speedup vs baseline: 20.0223x; 1.0852x over previous
import jax
import jax.numpy as jnp
from jax import lax
from jax.experimental import pallas as pl
from jax.experimental.pallas import tpu as pltpu, tpu_sc as plsc

N = 10000
D = 128
H = 16
C = 40
E = 320000

NC = 2
NS = 16
L = 16
NW = NC * NS

NPAD = 10240
RPT = NPAD // NS
CHUNK = 125
NCHUNK = 80
KINF = 4
NG = NCHUNK // KINF

BR = 1024
NBLK = NPAD // BR


def _sc_mesh():
    return plsc.VectorSubcoreMesh(core_axis_name="c", subcore_axis_name="s")


def _deg_call(edges, zrow):
    def body(e_hbm, z_hbm, dout0_hbm, dout1_hbm, din0_hbm, din1_hbm,
             sidx, didx, ones_v, sem, acc_out, acc_in):
        cid = lax.axis_index("c")
        sid = lax.axis_index("s")
        w = cid * NS + sid
        sl = pl.ds(sid * RPT, RPT)
        pltpu.sync_copy(z_hbm, acc_out.at[sl])
        pltpu.sync_copy(z_hbm, acc_in.at[sl])
        pltpu.sync_copy(e_hbm.at[0, w], sidx)
        pltpu.sync_copy(e_hbm.at[1, w], didx)
        for i in range(CHUNK // L + 1):
            o = min(i * L, CHUNK - L)
            ones_v[pl.ds(o, L)] = jnp.ones((L,), jnp.float32)
        plsc.subcore_barrier()

        DK = 4

        def fire(g):
            for b in range(DK):
                c = g * DK + b
                pltpu.async_copy(ones_v, acc_out.at[sidx.at[c]], sem, add=True)
                pltpu.async_copy(ones_v, acc_in.at[didx.at[c]], sem, add=True)

        def drain(g):
            for b in range(DK):
                c = g * DK + b
                pltpu.make_async_copy(ones_v, acc_out.at[sidx.at[c]],
                                      sem).wait()
                pltpu.make_async_copy(ones_v, acc_in.at[didx.at[c]],
                                      sem).wait()

        def step(g, carry):
            fire(g)

            @pl.when(g > 0)
            def _():
                drain(g - 1)
            return carry

        lax.fori_loop(0, NCHUNK // DK, step, 0)
        drain(NCHUNK // DK - 1)
        plsc.subcore_barrier()

        @pl.when(cid == 0)
        def _():
            pltpu.sync_copy(acc_out.at[sl], dout0_hbm.at[sl])
            pltpu.sync_copy(acc_in.at[sl], din0_hbm.at[sl])

        @pl.when(cid == 1)
        def _():
            pltpu.sync_copy(acc_out.at[sl], dout1_hbm.at[sl])
            pltpu.sync_copy(acc_in.at[sl], din1_hbm.at[sl])

    f = pl.kernel(
        body,
        out_type=[jax.ShapeDtypeStruct((NPAD,), jnp.float32)] * 4,
        mesh=_sc_mesh(),
        compiler_params=pltpu.CompilerParams(use_tc_tiling_on_sc=False),
        scratch_types=[
            pltpu.VMEM((NCHUNK, CHUNK), jnp.int32),
            pltpu.VMEM((NCHUNK, CHUNK), jnp.int32),
            pltpu.VMEM((CHUNK,), jnp.float32),
            pltpu.SemaphoreType.DMA,
            pltpu.VMEM_SHARED((NPAD,), jnp.float32),
            pltpu.VMEM_SHARED((NPAD,), jnp.float32),
        ],
    )
    return f(edges, zrow)


def _edge_call(h, edges, zrows, F):
    def body(h_hbm, e_hbm, z_hbm, part0_hbm, part1_hbm,
             sidx, didx, rows, semg, sems, acc, h_sp):
        cid = lax.axis_index("c")
        sid = lax.axis_index("s")
        w = cid * NS + sid
        sl = pl.ds(sid * RPT, RPT)
        pltpu.sync_copy(z_hbm, acc.at[sl])
        pltpu.sync_copy(h_hbm.at[sl], h_sp.at[sl])
        pltpu.sync_copy(e_hbm.at[0, w], sidx)
        pltpu.sync_copy(e_hbm.at[1, w], didx)
        plsc.subcore_barrier()

        def fire_gathers(g, pg):
            for b in range(KINF):
                pltpu.async_copy(
                    h_sp.at[sidx.at[g * KINF + b]], rows.at[pg, b], semg)

        def drain_gathers(g, pg):
            for b in range(KINF):
                pltpu.make_async_copy(
                    h_sp.at[sidx.at[g * KINF + b]], rows.at[pg, b],
                    semg).wait()

        def fire_scatters(g, pg):
            for b in range(KINF):
                pltpu.async_copy(
                    rows.at[pg, b], acc.at[didx.at[g * KINF + b]], sems,
                    add=True)

        def drain_scatters(g, pg):
            for b in range(KINF):
                pltpu.make_async_copy(
                    rows.at[pg, b], acc.at[didx.at[g * KINF + b]],
                    sems).wait()

        fire_gathers(0, 0)
        drain_gathers(0, 0)

        def step(g, carry):
            pg = lax.rem(g, 2)

            @pl.when(g > 0)
            def _():
                drain_scatters(g, 1 - pg)

            fire_scatters(g, pg)

            @pl.when(g < NG - 1)
            def _():
                fire_gathers(g + 1, 1 - pg)
                drain_gathers(g + 1, 1 - pg)

            return carry

        lax.fori_loop(0, NG, step, 0)
        drain_scatters(NG - 1, lax.rem(NG - 1, 2))
        plsc.subcore_barrier()

        @pl.when(cid == 0)
        def _():
            pltpu.sync_copy(acc.at[sl], part0_hbm.at[sl])

        @pl.when(cid == 1)
        def _():
            pltpu.sync_copy(acc.at[sl], part1_hbm.at[sl])

    f = pl.kernel(
        body,
        out_type=[jax.ShapeDtypeStruct((NPAD, F), jnp.float32)] * 2,
        mesh=_sc_mesh(),
        compiler_params=pltpu.CompilerParams(use_tc_tiling_on_sc=False),
        scratch_types=[
            pltpu.VMEM((NCHUNK, CHUNK), jnp.int32),
            pltpu.VMEM((NCHUNK, CHUNK), jnp.int32),
            pltpu.VMEM((2, KINF, CHUNK, F), jnp.float32),
            pltpu.SemaphoreType.DMA,
            pltpu.SemaphoreType.DMA,
            pltpu.VMEM_SHARED((NPAD, F), jnp.float32),
            pltpu.VMEM_SHARED((NPAD, F), jnp.float32),
        ],
    )
    return f(h, edges, zrows)


def _norms_call(do0, do1, di0, di1):
    def body(do0_ref, do1_ref, di0_ref, di1_ref, ns_ref, nd_ref):
        dsum_o = do0_ref[...] + do1_ref[...]
        dsum_i = di0_ref[...] + di1_ref[...]
        ns_ref[...] = jnp.where(
            dsum_o > 0, lax.rsqrt(jnp.maximum(dsum_o, 1.0)), 0.0)
        nd_ref[...] = jnp.where(
            dsum_i > 0, lax.rsqrt(jnp.maximum(dsum_i, 1.0)), 0.0)

    return pl.pallas_call(
        body,
        out_shape=[
            jax.ShapeDtypeStruct((NPAD,), jnp.float32),
            jax.ShapeDtypeStruct((NPAD,), jnp.float32),
        ],
    )(do0, do1, di0, di1)


def _tc1_call(features, W1, ns):
    def body(f_ref, w_ref, ns_ref, h1_ref):
        nsv = ns_ref[...].reshape(BR, 1)
        m = jnp.dot(f_ref[...], w_ref[...], preferred_element_type=jnp.float32)
        h1_ref[...] = m * nsv

    return pl.pallas_call(
        body,
        grid=(NBLK,),
        in_specs=[
            pl.BlockSpec((BR, D), lambda i: (i, 0)),
            pl.BlockSpec((D, H), lambda i: (0, 0)),
            pl.BlockSpec((BR,), lambda i: (i,)),
        ],
        out_specs=pl.BlockSpec((BR, H), lambda i: (i, 0)),
        out_shape=jax.ShapeDtypeStruct((NPAD, H), jnp.float32),
    )(features, W1, ns)


def _tc2_call(p0, p1, nd, ns, b1r, W2):
    def body(p0_ref, p1_ref, nd_ref, ns_ref, b1_ref, w2_ref, h2_ref):
        pf = p0_ref[...] + p1_ref[...]
        parts = [pf[:, a * H:(a + 1) * H] for a in range(128 // H)]
        agg = jnp.stack(parts, axis=1).reshape(BR, H)
        ndv = nd_ref[...].reshape(BR, 1)
        nsv = ns_ref[...].reshape(BR, 1)
        x = jnp.maximum(agg * ndv + b1_ref[...], 0.0)
        h2 = jnp.dot(x, w2_ref[...], preferred_element_type=jnp.float32)
        h2_ref[...] = h2 * nsv

    nrow = BR // 128
    return pl.pallas_call(
        body,
        grid=(NBLK,),
        in_specs=[
            pl.BlockSpec((BR * H // 128, 128), lambda i: (i, 0)),
            pl.BlockSpec((BR * H // 128, 128), lambda i: (i, 0)),
            pl.BlockSpec((BR,), lambda i: (i,)),
            pl.BlockSpec((BR,), lambda i: (i,)),
            pl.BlockSpec((1, H), lambda i: (0, 0)),
            pl.BlockSpec((H, C), lambda i: (0, 0)),
        ],
        out_specs=pl.BlockSpec((BR, C), lambda i: (i, 0)),
        out_shape=jax.ShapeDtypeStruct((NPAD, C), jnp.float32),
    )(p0, p1, nd, ns, b1r, W2)


def _tc3_call(q0, q1, ndrep, b2p):
    def body(q0_ref, q1_ref, ndr_ref, b2_ref, o_ref):
        o_ref[...] = (q0_ref[...] + q1_ref[...]) * ndr_ref[...] + b2_ref[...]

    rows = BR * C // 128
    return pl.pallas_call(
        body,
        grid=(NBLK,),
        in_specs=[
            pl.BlockSpec((rows, 128), lambda i: (i, 0)),
            pl.BlockSpec((rows, 128), lambda i: (i, 0)),
            pl.BlockSpec((rows, 128), lambda i: (i, 0)),
            pl.BlockSpec((rows, 128), lambda i: (0, 0)),
        ],
        out_specs=pl.BlockSpec((rows, 128), lambda i: (i, 0)),
        out_shape=jax.ShapeDtypeStruct((N * C // 128, 128), jnp.float32),
    )(q0, q1, ndrep, b2p)


def kernel(features, edge_index, W1, b1, W2, b2):
    edges = edge_index.reshape(2, NW, NCHUNK, CHUNK)

    do0, do1, di0, di1 = _deg_call(edges, jnp.zeros((RPT,), jnp.float32))
    ns, nd = _norms_call(do0, do1, di0, di1)
    h1 = _tc1_call(features, W1, ns)
    p0, p1 = _edge_call(h1, edges, jnp.zeros((RPT, H), jnp.float32), H)
    h2 = _tc2_call(p0.reshape(NPAD * H // 128, 128),
                   p1.reshape(NPAD * H // 128, 128),
                   nd, ns, b1.reshape(1, H), W2)
    q0, q1 = _edge_call(h2, edges, jnp.zeros((RPT, C), jnp.float32), C)
    ndrep = jnp.repeat(nd, C).reshape(NPAD * C // 128, 128)
    b2p = jnp.tile(b2, BR).reshape(BR * C // 128, 128)
    outp = _tc3_call(q0.reshape(NPAD * C // 128, 128),
                     q1.reshape(NPAD * C // 128, 128),
                     ndrep, b2p)
    return outp.reshape(N, C)


# speedup vs baseline: 20.1940x; 1.0086x over previous
import jax
import jax.numpy as jnp
from jax import lax
from jax.experimental import pallas as pl
from jax.experimental.pallas import tpu as pltpu, tpu_sc as plsc

N = 10000
D = 128
H = 16
C = 40
E = 320000

NC = 2
NS = 16
L = 16
NW = NC * NS

NPAD = 10240
RPT = NPAD // NS
CHUNK = 125
NCHUNK = 80

BR = 1024
NBLK = NPAD // BR


def _sc_mesh():
    return plsc.VectorSubcoreMesh(core_axis_name="c", subcore_axis_name="s")


def _deg_call(edges, zrow):
    def body(e_hbm, z_hbm, dout0_hbm, dout1_hbm, din0_hbm, din1_hbm,
             sidx, didx, ones_v, sem, acc_out, acc_in):
        cid = lax.axis_index("c")
        sid = lax.axis_index("s")
        w = cid * NS + sid
        sl = pl.ds(sid * RPT, RPT)
        pltpu.sync_copy(z_hbm, acc_out.at[sl])
        pltpu.sync_copy(z_hbm, acc_in.at[sl])
        pltpu.sync_copy(e_hbm.at[0, w], sidx)
        pltpu.sync_copy(e_hbm.at[1, w], didx)
        for i in range(CHUNK // L + 1):
            o = min(i * L, CHUNK - L)
            ones_v[pl.ds(o, L)] = jnp.ones((L,), jnp.float32)
        plsc.subcore_barrier()

        DK = 4

        def fire(g):
            for b in range(DK):
                c = g * DK + b
                pltpu.async_copy(ones_v, acc_out.at[sidx.at[c]], sem, add=True)
                pltpu.async_copy(ones_v, acc_in.at[didx.at[c]], sem, add=True)

        def drain(g):
            for b in range(DK):
                c = g * DK + b
                pltpu.make_async_copy(ones_v, acc_out.at[sidx.at[c]],
                                      sem).wait()
                pltpu.make_async_copy(ones_v, acc_in.at[didx.at[c]],
                                      sem).wait()

        def step(g, carry):
            fire(g)

            @pl.when(g > 0)
            def _():
                drain(g - 1)
            return carry

        lax.fori_loop(0, NCHUNK // DK, step, 0)
        drain(NCHUNK // DK - 1)
        plsc.subcore_barrier()

        @pl.when(cid == 0)
        def _():
            pltpu.sync_copy(acc_out.at[sl], dout0_hbm.at[sl])
            pltpu.sync_copy(acc_in.at[sl], din0_hbm.at[sl])

        @pl.when(cid == 1)
        def _():
            pltpu.sync_copy(acc_out.at[sl], dout1_hbm.at[sl])
            pltpu.sync_copy(acc_in.at[sl], din1_hbm.at[sl])

    f = pl.kernel(
        body,
        out_type=[jax.ShapeDtypeStruct((NPAD,), jnp.float32)] * 4,
        mesh=_sc_mesh(),
        compiler_params=pltpu.CompilerParams(use_tc_tiling_on_sc=False),
        scratch_types=[
            pltpu.VMEM((NCHUNK, CHUNK), jnp.int32),
            pltpu.VMEM((NCHUNK, CHUNK), jnp.int32),
            pltpu.VMEM((CHUNK,), jnp.float32),
            pltpu.SemaphoreType.DMA,
            pltpu.VMEM_SHARED((NPAD,), jnp.float32),
            pltpu.VMEM_SHARED((NPAD,), jnp.float32),
        ],
    )
    return f(edges, zrow)


def _edge_call(h, edges, zrows, F):
    KINF = 8 if F == H else 4
    NG = NCHUNK // KINF

    def body(h_hbm, e_hbm, z_hbm, part0_hbm, part1_hbm,
             sidx, didx, rows, semg, sems, acc, h_sp):
        cid = lax.axis_index("c")
        sid = lax.axis_index("s")
        w = cid * NS + sid
        sl = pl.ds(sid * RPT, RPT)
        pltpu.sync_copy(z_hbm, acc.at[sl])
        pltpu.sync_copy(h_hbm.at[sl], h_sp.at[sl])
        pltpu.sync_copy(e_hbm.at[0, w], sidx)
        pltpu.sync_copy(e_hbm.at[1, w], didx)
        plsc.subcore_barrier()

        def fire_gathers(g, pg):
            for b in range(KINF):
                pltpu.async_copy(
                    h_sp.at[sidx.at[g * KINF + b]], rows.at[pg, b], semg)

        def drain_gathers(g, pg):
            for b in range(KINF):
                pltpu.make_async_copy(
                    h_sp.at[sidx.at[g * KINF + b]], rows.at[pg, b],
                    semg).wait()

        def fire_scatters(g, pg):
            for b in range(KINF):
                pltpu.async_copy(
                    rows.at[pg, b], acc.at[didx.at[g * KINF + b]], sems,
                    add=True)

        def drain_scatters(g, pg):
            for b in range(KINF):
                pltpu.make_async_copy(
                    rows.at[pg, b], acc.at[didx.at[g * KINF + b]],
                    sems).wait()

        fire_gathers(0, 0)
        drain_gathers(0, 0)

        def step(g, carry):
            pg = lax.rem(g, 2)

            @pl.when(g > 0)
            def _():
                drain_scatters(g, 1 - pg)

            fire_scatters(g, pg)

            @pl.when(g < NG - 1)
            def _():
                fire_gathers(g + 1, 1 - pg)
                drain_gathers(g + 1, 1 - pg)

            return carry

        lax.fori_loop(0, NG, step, 0)
        drain_scatters(NG - 1, lax.rem(NG - 1, 2))
        plsc.subcore_barrier()

        @pl.when(cid == 0)
        def _():
            pltpu.sync_copy(acc.at[sl], part0_hbm.at[sl])

        @pl.when(cid == 1)
        def _():
            pltpu.sync_copy(acc.at[sl], part1_hbm.at[sl])

    f = pl.kernel(
        body,
        out_type=[jax.ShapeDtypeStruct((NPAD, F), jnp.float32)] * 2,
        mesh=_sc_mesh(),
        compiler_params=pltpu.CompilerParams(use_tc_tiling_on_sc=False),
        scratch_types=[
            pltpu.VMEM((NCHUNK, CHUNK), jnp.int32),
            pltpu.VMEM((NCHUNK, CHUNK), jnp.int32),
            pltpu.VMEM((2, KINF, CHUNK, F), jnp.float32),
            pltpu.SemaphoreType.DMA,
            pltpu.SemaphoreType.DMA,
            pltpu.VMEM_SHARED((NPAD, F), jnp.float32),
            pltpu.VMEM_SHARED((NPAD, F), jnp.float32),
        ],
    )
    return f(h, edges, zrows)


def _norms_call(do0, do1, di0, di1):
    def body(do0_ref, do1_ref, di0_ref, di1_ref, ns_ref, nd_ref):
        dsum_o = do0_ref[...] + do1_ref[...]
        dsum_i = di0_ref[...] + di1_ref[...]
        ns_ref[...] = jnp.where(
            dsum_o > 0, lax.rsqrt(jnp.maximum(dsum_o, 1.0)), 0.0)
        nd_ref[...] = jnp.where(
            dsum_i > 0, lax.rsqrt(jnp.maximum(dsum_i, 1.0)), 0.0)

    return pl.pallas_call(
        body,
        out_shape=[
            jax.ShapeDtypeStruct((NPAD,), jnp.float32),
            jax.ShapeDtypeStruct((NPAD,), jnp.float32),
        ],
    )(do0, do1, di0, di1)


def _tc1_call(features, W1, ns):
    def body(f_ref, w_ref, ns_ref, h1_ref):
        nsv = ns_ref[...].reshape(BR, 1)
        m = jnp.dot(f_ref[...], w_ref[...], preferred_element_type=jnp.float32)
        h1_ref[...] = m * nsv

    return pl.pallas_call(
        body,
        grid=(NBLK,),
        in_specs=[
            pl.BlockSpec((BR, D), lambda i: (i, 0)),
            pl.BlockSpec((D, H), lambda i: (0, 0)),
            pl.BlockSpec((BR,), lambda i: (i,)),
        ],
        out_specs=pl.BlockSpec((BR, H), lambda i: (i, 0)),
        out_shape=jax.ShapeDtypeStruct((NPAD, H), jnp.float32),
    )(features, W1, ns)


def _tc2_call(p0, p1, nd, ns, b1r, W2):
    def body(p0_ref, p1_ref, nd_ref, ns_ref, b1_ref, w2_ref, h2_ref):
        pf = p0_ref[...] + p1_ref[...]
        parts = [pf[:, a * H:(a + 1) * H] for a in range(128 // H)]
        agg = jnp.stack(parts, axis=1).reshape(BR, H)
        ndv = nd_ref[...].reshape(BR, 1)
        nsv = ns_ref[...].reshape(BR, 1)
        x = jnp.maximum(agg * ndv + b1_ref[...], 0.0)
        h2 = jnp.dot(x, w2_ref[...], preferred_element_type=jnp.float32)
        h2_ref[...] = h2 * nsv

    return pl.pallas_call(
        body,
        grid=(NBLK,),
        in_specs=[
            pl.BlockSpec((BR * H // 128, 128), lambda i: (i, 0)),
            pl.BlockSpec((BR * H // 128, 128), lambda i: (i, 0)),
            pl.BlockSpec((BR,), lambda i: (i,)),
            pl.BlockSpec((BR,), lambda i: (i,)),
            pl.BlockSpec((1, H), lambda i: (0, 0)),
            pl.BlockSpec((H, C), lambda i: (0, 0)),
        ],
        out_specs=pl.BlockSpec((BR, C), lambda i: (i, 0)),
        out_shape=jax.ShapeDtypeStruct((NPAD, C), jnp.float32),
    )(p0, p1, nd, ns, b1r, W2)


def _tc3_call(q0, q1, ndrep, b2p):
    def body(q0_ref, q1_ref, ndr_ref, b2_ref, o_ref):
        o_ref[...] = (q0_ref[...] + q1_ref[...]) * ndr_ref[...] + b2_ref[...]

    rows = BR * C // 128
    return pl.pallas_call(
        body,
        grid=(NBLK,),
        in_specs=[
            pl.BlockSpec((rows, 128), lambda i: (i, 0)),
            pl.BlockSpec((rows, 128), lambda i: (i, 0)),
            pl.BlockSpec((rows, 128), lambda i: (i, 0)),
            pl.BlockSpec((rows, 128), lambda i: (0, 0)),
        ],
        out_specs=pl.BlockSpec((rows, 128), lambda i: (i, 0)),
        out_shape=jax.ShapeDtypeStruct((N * C // 128, 128), jnp.float32),
    )(q0, q1, ndrep, b2p)


def kernel(features, edge_index, W1, b1, W2, b2):
    edges = edge_index.reshape(2, NW, NCHUNK, CHUNK)

    do0, do1, di0, di1 = _deg_call(edges, jnp.zeros((RPT,), jnp.float32))
    ns, nd = _norms_call(do0, do1, di0, di1)
    h1 = _tc1_call(features, W1, ns)
    p0, p1 = _edge_call(h1, edges, jnp.zeros((RPT, H), jnp.float32), H)
    h2 = _tc2_call(p0.reshape(NPAD * H // 128, 128),
                   p1.reshape(NPAD * H // 128, 128),
                   nd, ns, b1.reshape(1, H), W2)
    q0, q1 = _edge_call(h2, edges, jnp.zeros((RPT, C), jnp.float32), C)
    ndrep = jnp.repeat(nd, C).reshape(NPAD * C // 128, 128)
    b2p = jnp.tile(b2, BR).reshape(BR * C // 128, 128)
    outp = _tc3_call(q0.reshape(NPAD * C // 128, 128),
                     q1.reshape(NPAD * C // 128, 128),
                     ndrep, b2p)
    return outp.reshape(N, C)
